# baseline (speedup 1.0000x reference)
import functools

import jax
import jax.numpy as jnp
import numpy as np
from jax import lax
from jax.experimental import pallas as pl
from jax.experimental.pallas import tpu as pltpu
from jax.experimental.pallas import tpu_sc as plsc

_FIELD_DIMS = [38462] * 26
_NUM_FIELDS = 26
_VOCAB = sum(_FIELD_DIMS)
_EMBED_DIM = 16
_BATCH = 16384
_MLP_IN = _NUM_FIELDS * _EMBED_DIM
_OFFSETS = np.concatenate(([0], np.cumsum(_FIELD_DIMS)[:-1])).astype(np.int32)

_NW = 32
_N_FC = _BATCH * _NUM_FIELDS
_PW_F = _N_FC // _NW
_CH_F = 3328
_BS = 1024
_NBLK = _BATCH // _BS

_VB = 4096
_GVOC = 8 * 38462
_BSTART = [(j * _GVOC) // _VB for j in range(4)]
_BEND = [-(-min((j + 1) * _GVOC, _VOCAB) // _VB) for j in range(4)]
_NBJ = [_BEND[j] - _BSTART[j] for j in range(4)]
_NE_J = _BATCH * 8
_PW_E = _NE_J // _NW
_CH_E = 4096


def _fold_body(in_ref, out_ref, scr_ref):
    scr_ref[:, 0:16] = in_ref[...].T
    for s in range(8):
        out_ref[:, s * 16:(s + 1) * 16] = scr_ref[pl.Slice(s, _VB // 8, 8), 0:16]


def _fold_slab(embT, j):
    bs = _BSTART[j]
    return pl.pallas_call(
        _fold_body,
        grid=(_NBJ[j],),
        in_specs=[pl.BlockSpec((16, _VB), lambda i: (0, bs + i))],
        out_specs=pl.BlockSpec((_VB // 8, 128), lambda i: (i, 0)),
        out_shape=jax.ShapeDtypeStruct((_NBJ[j] * _VB // 8, 128), jnp.float32),
        scratch_shapes=[pltpu.VMEM((_VB, 128), jnp.float32)],
    )(embT)


_NBF = (_VOCAB + _VB - 1) // _VB


def _fc_fold_body(in_ref, out_ref):
    out_ref[...] = in_ref[...].reshape(_VB // 128, 128)


def _fc_fold(fcT):
    return pl.pallas_call(
        _fc_fold_body,
        grid=(_NBF,),
        in_specs=[pl.BlockSpec((1, _VB), lambda i: (0, i))],
        out_specs=pl.BlockSpec((_VB // 128, 128), lambda i: (i, 0)),
        out_shape=jax.ShapeDtypeStruct((_NBF * _VB // 128, 128), jnp.float32),
    )(fcT)


def _sc_gather_slab(table, idxp_all, j, with_fc, fc1=None, idxf=None):
    mesh = plsc.VectorSubcoreMesh(core_axis_name="c", subcore_axis_name="s")
    out_type = [jax.ShapeDtypeStruct((_NE_J, _EMBED_DIM), jnp.float32)]
    scratch = [
        pltpu.VMEM((_CH_E,), jnp.int32),
        pltpu.VMEM((_CH_E, _EMBED_DIM), jnp.float32),
        pltpu.SemaphoreType.DMA,
    ]
    if with_fc:
        out_type.append(jax.ShapeDtypeStruct((_N_FC,), jnp.float32))
        scratch += [
            pltpu.VMEM((_CH_F,), jnp.int32),
            pltpu.VMEM((_CH_F,), jnp.float32),
            pltpu.SemaphoreType.DMA,
        ]

    @functools.partial(
        pl.kernel,
        out_type=tuple(out_type),
        name="sc_gather_fc" if with_fc else "sc_gather",
        mesh=mesh,
        scratch_types=scratch,
        compiler_params=pltpu.CompilerParams(use_tc_tiling_on_sc=False),
    )
    def k(*refs):
        if with_fc:
            (emb_hbm, fc_hbm, idxp_hbm, idxf_hbm, ex_hbm, fcg_hbm,
             idxe_v, rows_v, s1, idxf_v, fcr_v, s2) = refs
        else:
            emb_hbm, idxp_hbm, ex_hbm, idxe_v, rows_v, s1 = refs
        wid = lax.axis_index("s") * 2 + lax.axis_index("c")
        st = j * _NE_J + wid * _PW_E
        pltpu.sync_copy(idxp_hbm.at[pl.ds(st, _CH_E)], idxe_v)
        pltpu.async_copy(emb_hbm.at[idxe_v], rows_v, s1).wait()
        pltpu.sync_copy(rows_v, ex_hbm.at[pl.ds(wid * _PW_E, _CH_E)])
        if with_fc:
            def body_f(t, carry):
                stf = wid * _PW_F + t * _CH_F
                pltpu.sync_copy(idxf_hbm.at[pl.ds(stf, _CH_F)], idxf_v)
                pltpu.async_copy(fc_hbm.at[idxf_v], fcr_v, s2).wait()
                pltpu.sync_copy(fcr_v, fcg_hbm.at[pl.ds(stf, _CH_F)])
                return carry
            lax.fori_loop(0, _PW_F // _CH_F, body_f, 0)

    if with_fc:
        return k(table, fc1, idxp_all, idxf)
    return k(table, idxp_all)[0]


def _tc_body(e0_ref, e1_ref, e2_ref, e3_ref, fcg_ref, sp_ref, w1_ref, b1_ref,
             w2_ref, b2_ref, w3_ref, cb_ref, out_ref):
    exs = [e0_ref[...], e1_ref[...], e2_ref[...], e3_ref[...]]
    rowsum = jnp.zeros((_BS, _EMBED_DIM), jnp.float32)
    y = jnp.zeros((_BS, 128), jnp.float32)
    ssq = jnp.zeros((_BS,), jnp.float32)
    for j in range(4):
        exj = exs[j]
        y = y + jnp.dot(exj, w1_ref[j * 128:(j + 1) * 128, :],
                        preferred_element_type=jnp.float32)
        rowsum = rowsum + jnp.dot(exj, sp_ref[j * 128:(j + 1) * 128, :],
                                  preferred_element_type=jnp.float32)
        if j < 3:
            ssq = ssq + jnp.sum(exj * exj, axis=1)
        else:
            ex3 = exj[:, :32]
            ssq = ssq + jnp.sum(ex3 * ex3, axis=1)
    fm = 0.5 * (jnp.sum(rowsum * rowsum, axis=1) - ssq)
    lin = jnp.sum(fcg_ref[...], axis=1)
    h1 = jnp.maximum(y + b1_ref[...], 0.0)
    h2 = jnp.maximum(
        jnp.dot(h1, w2_ref[...], preferred_element_type=jnp.float32)
        + b2_ref[...], 0.0)
    mlp = jnp.sum(h2 * w3_ref[...], axis=1)
    out_ref[...] = lin + fm + mlp + cb_ref[0, 0]


_SEL = np.zeros((512, _EMBED_DIM), np.float32)
for _f in range(_NUM_FIELDS):
    for _d in range(_EMBED_DIM):
        _SEL[(_f // 8) * 128 + (_f % 8) * 16 + _d, _d] = 1.0


def _tc_compute(exs, fcg, W1p, b1, W2, b2, w3, cb):
    eb = pl.BlockSpec((_BS * 8 // 128 * 16, 128), lambda i: (i, 0))
    return pl.pallas_call(
        _tc_body,
        grid=(_NBLK,),
        in_specs=[
            pl.BlockSpec((_BS, 128), lambda i: (i, 0)),
            pl.BlockSpec((_BS, 128), lambda i: (i, 0)),
            pl.BlockSpec((_BS, 128), lambda i: (i, 0)),
            pl.BlockSpec((_BS, 128), lambda i: (i, 0)),
            pl.BlockSpec((_BS, _NUM_FIELDS), lambda i: (i, 0)),
            pl.BlockSpec((512, _EMBED_DIM), lambda i: (0, 0)),
            pl.BlockSpec((512, 128), lambda i: (0, 0)),
            pl.BlockSpec((1, 128), lambda i: (0, 0)),
            pl.BlockSpec((128, 64), lambda i: (0, 0)),
            pl.BlockSpec((1, 64), lambda i: (0, 0)),
            pl.BlockSpec((1, 64), lambda i: (0, 0)),
            pl.BlockSpec((1, 1), lambda i: (0, 0)),
        ],
        out_specs=pl.BlockSpec((_BS,), lambda i: (i,)),
        out_shape=jax.ShapeDtypeStruct((_BATCH,), jnp.float32),
    )(*exs, fcg, jnp.asarray(_SEL), W1p, b1, W2, b2, w3, cb)


def kernel(x, emb, fc, bias, W1, b1, W2, b2, W3, b3):
    idx = x.astype(jnp.int32) + jnp.asarray(_OFFSETS, jnp.int32)[None, :]
    idxf = idx.reshape(-1)
    embT = emb.T
    fc1 = _fc_fold(fc.T).reshape(-1)
    pieces = []
    for j in range(4):
        nf = min(8, _NUM_FIELDS - j * 8)
        cols = idx[:, j * 8:j * 8 + nf] - (_BSTART[j] * _VB)
        if nf < 8:
            reps = [cols[:, i % nf:i % nf + 1] for i in range(8 - nf)]
            cols = jnp.concatenate([cols] + reps, axis=1)
        pieces.append(cols.reshape(-1))
    idxp_all = jnp.concatenate(pieces)
    exs = []
    fcg_flat = None
    for j in range(4):
        table_j = _fold_slab(embT, j).reshape(_NBJ[j] * _VB, _EMBED_DIM)
        if j == 0:
            ex_j, fcg_flat = _sc_gather_slab(table_j, idxp_all, j, True,
                                             fc1, idxf)
        else:
            ex_j = _sc_gather_slab(table_j, idxp_all, j, False)
        exs.append(ex_j.reshape(_NE_J // 8, 128))
    fcg = fcg_flat.reshape(_BATCH, _NUM_FIELDS)
    W1p = jnp.concatenate(
        [W1.reshape(_NUM_FIELDS, _EMBED_DIM, 128),
         jnp.zeros((32 - _NUM_FIELDS, _EMBED_DIM, 128), jnp.float32)],
        axis=0).reshape(512, 128)
    cb = (bias + b3).reshape(1, 1)
    return _tc_compute(exs, fcg, W1p, b1.reshape(1, 128), W2,
                       b2.reshape(1, 64), W3.reshape(1, 64), cb)


# speedup vs baseline: 1.1906x; 1.1906x over previous
import functools

import jax
import jax.numpy as jnp
import numpy as np
from jax import lax
from jax.experimental import pallas as pl
from jax.experimental.pallas import tpu as pltpu
from jax.experimental.pallas import tpu_sc as plsc

_FIELD_DIMS = [38462] * 26
_NUM_FIELDS = 26
_VOCAB = sum(_FIELD_DIMS)
_EMBED_DIM = 16
_BATCH = 16384
_MLP_IN = _NUM_FIELDS * _EMBED_DIM
_OFFSETS = np.concatenate(([0], np.cumsum(_FIELD_DIMS)[:-1])).astype(np.int32)

_NW = 32
_N_FC = _BATCH * _NUM_FIELDS
_PW_F = _N_FC // _NW
_CH_F = 3328
_BS = 1024
_NBLK = _BATCH // _BS

_VB = 4096
_GVOC = 8 * 38462
_BSTART = [(j * _GVOC) // _VB for j in range(4)]
_BEND = [-(-min((j + 1) * _GVOC, _VOCAB) // _VB) for j in range(4)]
_NBJ = [_BEND[j] - _BSTART[j] for j in range(4)]
_NE_J = _BATCH * 8
_PW_E = _NE_J // _NW
_CH_E = 4096


def _fold_body(in_ref, out_ref, scr_ref):
    scr_ref[:, 0:16] = in_ref[...].T
    for s in range(8):
        out_ref[:, s * 16:(s + 1) * 16] = scr_ref[pl.Slice(s, _VB // 8, 8), 0:16]


def _fold_slab(embT, j):
    bs = _BSTART[j]
    return pl.pallas_call(
        _fold_body,
        grid=(_NBJ[j],),
        in_specs=[pl.BlockSpec((16, _VB), lambda i: (0, bs + i))],
        out_specs=pl.BlockSpec((_VB // 8, 128), lambda i: (i, 0)),
        out_shape=jax.ShapeDtypeStruct((_NBJ[j] * _VB // 8, 128), jnp.float32),
        scratch_shapes=[pltpu.VMEM((_VB, 128), jnp.float32)],
    )(embT)


_FCB = 65536
_NBF = (_VOCAB + _FCB - 1) // _FCB


def _fc_fold_body(in_ref, out_ref):
    out_ref[...] = in_ref[...].reshape(_FCB // 128, 128)


def _fc_fold(fcT):
    return pl.pallas_call(
        _fc_fold_body,
        grid=(_NBF,),
        in_specs=[pl.BlockSpec((1, _FCB), lambda i: (0, i))],
        out_specs=pl.BlockSpec((_FCB // 128, 128), lambda i: (i, 0)),
        out_shape=jax.ShapeDtypeStruct((_NBF * _FCB // 128, 128), jnp.float32),
    )(fcT)


def _sc_gather_slab(table, idxp_all, j, with_fc, fc1=None, idxf=None):
    mesh = plsc.VectorSubcoreMesh(core_axis_name="c", subcore_axis_name="s")
    out_type = [jax.ShapeDtypeStruct((_NE_J, _EMBED_DIM), jnp.float32)]
    scratch = [
        pltpu.VMEM((_CH_E,), jnp.int32),
        pltpu.VMEM((_CH_E, _EMBED_DIM), jnp.float32),
        pltpu.SemaphoreType.DMA,
    ]
    if with_fc:
        out_type.append(jax.ShapeDtypeStruct((_N_FC,), jnp.float32))
        scratch += [
            pltpu.VMEM((_CH_F,), jnp.int32),
            pltpu.VMEM((_CH_F,), jnp.float32),
            pltpu.SemaphoreType.DMA,
        ]

    @functools.partial(
        pl.kernel,
        out_type=tuple(out_type),
        name="sc_gather_fc" if with_fc else "sc_gather",
        mesh=mesh,
        scratch_types=scratch,
        compiler_params=pltpu.CompilerParams(use_tc_tiling_on_sc=False),
    )
    def k(*refs):
        if with_fc:
            (emb_hbm, fc_hbm, idxp_hbm, idxf_hbm, ex_hbm, fcg_hbm,
             idxe_v, rows_v, s1, idxf_v, fcr_v, s2) = refs
        else:
            emb_hbm, idxp_hbm, ex_hbm, idxe_v, rows_v, s1 = refs
        wid = lax.axis_index("s") * 2 + lax.axis_index("c")
        st = j * _NE_J + wid * _PW_E
        pltpu.sync_copy(idxp_hbm.at[pl.ds(st, _CH_E)], idxe_v)
        pltpu.async_copy(emb_hbm.at[idxe_v], rows_v, s1).wait()
        pltpu.sync_copy(rows_v, ex_hbm.at[pl.ds(wid * _PW_E, _CH_E)])
        if with_fc:
            def body_f(t, carry):
                stf = wid * _PW_F + t * _CH_F
                pltpu.sync_copy(idxf_hbm.at[pl.ds(stf, _CH_F)], idxf_v)
                pltpu.async_copy(fc_hbm.at[idxf_v], fcr_v, s2).wait()
                pltpu.sync_copy(fcr_v, fcg_hbm.at[pl.ds(stf, _CH_F)])
                return carry
            lax.fori_loop(0, _PW_F // _CH_F, body_f, 0)

    if with_fc:
        return k(table, fc1, idxp_all, idxf)
    return k(table, idxp_all)[0]


def _tc_body(e0_ref, e1_ref, e2_ref, e3_ref, fcg_ref, sp_ref, w1_ref, b1_ref,
             w2_ref, b2_ref, w3_ref, cb_ref, out_ref):
    exs = [e0_ref[...], e1_ref[...], e2_ref[...], e3_ref[...]]
    rowsum = jnp.zeros((_BS, _EMBED_DIM), jnp.float32)
    y = jnp.zeros((_BS, 128), jnp.float32)
    ssq = jnp.zeros((_BS,), jnp.float32)
    for j in range(4):
        exj = exs[j]
        y = y + jnp.dot(exj, w1_ref[j * 128:(j + 1) * 128, :],
                        preferred_element_type=jnp.float32)
        rowsum = rowsum + jnp.dot(exj, sp_ref[j * 128:(j + 1) * 128, :],
                                  preferred_element_type=jnp.float32)
        if j < 3:
            ssq = ssq + jnp.sum(exj * exj, axis=1)
        else:
            ex3 = exj[:, :32]
            ssq = ssq + jnp.sum(ex3 * ex3, axis=1)
    fm = 0.5 * (jnp.sum(rowsum * rowsum, axis=1) - ssq)
    lin = jnp.sum(fcg_ref[...], axis=1)
    h1 = jnp.maximum(y + b1_ref[...], 0.0)
    h2 = jnp.maximum(
        jnp.dot(h1, w2_ref[...], preferred_element_type=jnp.float32)
        + b2_ref[...], 0.0)
    mlp = jnp.sum(h2 * w3_ref[...], axis=1)
    out_ref[...] = lin + fm + mlp + cb_ref[0, 0]


_SEL = np.zeros((512, _EMBED_DIM), np.float32)
for _f in range(_NUM_FIELDS):
    for _d in range(_EMBED_DIM):
        _SEL[(_f // 8) * 128 + (_f % 8) * 16 + _d, _d] = 1.0


def _tc_compute(exs, fcg, W1p, b1, W2, b2, w3, cb):
    eb = pl.BlockSpec((_BS * 8 // 128 * 16, 128), lambda i: (i, 0))
    return pl.pallas_call(
        _tc_body,
        grid=(_NBLK,),
        in_specs=[
            pl.BlockSpec((_BS, 128), lambda i: (i, 0)),
            pl.BlockSpec((_BS, 128), lambda i: (i, 0)),
            pl.BlockSpec((_BS, 128), lambda i: (i, 0)),
            pl.BlockSpec((_BS, 128), lambda i: (i, 0)),
            pl.BlockSpec((_BS, _NUM_FIELDS), lambda i: (i, 0)),
            pl.BlockSpec((512, _EMBED_DIM), lambda i: (0, 0)),
            pl.BlockSpec((512, 128), lambda i: (0, 0)),
            pl.BlockSpec((1, 128), lambda i: (0, 0)),
            pl.BlockSpec((128, 64), lambda i: (0, 0)),
            pl.BlockSpec((1, 64), lambda i: (0, 0)),
            pl.BlockSpec((1, 64), lambda i: (0, 0)),
            pl.BlockSpec((1, 1), lambda i: (0, 0)),
        ],
        out_specs=pl.BlockSpec((_BS,), lambda i: (i,)),
        out_shape=jax.ShapeDtypeStruct((_BATCH,), jnp.float32),
    )(*exs, fcg, jnp.asarray(_SEL), W1p, b1, W2, b2, w3, cb)


def kernel(x, emb, fc, bias, W1, b1, W2, b2, W3, b3):
    idx = x.astype(jnp.int32) + jnp.asarray(_OFFSETS, jnp.int32)[None, :]
    idxf = idx.reshape(-1)
    embT = emb.T
    fc1 = _fc_fold(fc.T).reshape(-1)
    padded = jnp.concatenate(
        [idx, idx[:, 24:26], idx[:, 24:26], idx[:, 24:26]], axis=1)
    slot_off = np.zeros((32,), np.int32)
    for c in range(32):
        slot_off[c] = _BSTART[min(c, 31) // 8 if c < 26 else 3] * _VB
    padded = padded - jnp.asarray(slot_off)[None, :]
    idxp_all = padded.reshape(_BATCH, 4, 8).transpose(1, 0, 2).reshape(-1)
    exs = []
    fcg_flat = None
    for j in range(4):
        table_j = _fold_slab(embT, j).reshape(_NBJ[j] * _VB, _EMBED_DIM)
        if j == 0:
            ex_j, fcg_flat = _sc_gather_slab(table_j, idxp_all, j, True,
                                             fc1, idxf)
        else:
            ex_j = _sc_gather_slab(table_j, idxp_all, j, False)
        exs.append(ex_j.reshape(_NE_J // 8, 128))
    fcg = fcg_flat.reshape(_BATCH, _NUM_FIELDS)
    W1p = jnp.concatenate(
        [W1.reshape(_NUM_FIELDS, _EMBED_DIM, 128),
         jnp.zeros((32 - _NUM_FIELDS, _EMBED_DIM, 128), jnp.float32)],
        axis=0).reshape(512, 128)
    cb = (bias + b3).reshape(1, 1)
    return _tc_compute(exs, fcg, W1p, b1.reshape(1, 128), W2,
                       b2.reshape(1, 64), W3.reshape(1, 64), cb)


# speedup vs baseline: 1.2179x; 1.0229x over previous
import functools

import jax
import jax.numpy as jnp
import numpy as np
from jax import lax
from jax.experimental import pallas as pl
from jax.experimental.pallas import tpu as pltpu
from jax.experimental.pallas import tpu_sc as plsc

_FIELD_DIMS = [38462] * 26
_NUM_FIELDS = 26
_VOCAB = sum(_FIELD_DIMS)
_EMBED_DIM = 16
_BATCH = 16384
_MLP_IN = _NUM_FIELDS * _EMBED_DIM
_OFFSETS = np.concatenate(([0], np.cumsum(_FIELD_DIMS)[:-1])).astype(np.int32)

_NW = 32
_N_FC = _BATCH * _NUM_FIELDS
_PW_F = _N_FC // _NW
_CH_F = 3328
_BS = 1024
_NBLK = _BATCH // _BS

_VB = 8192
_GVOC = 8 * 38462
_BSTART = [(j * _GVOC) // _VB for j in range(4)]
_BEND = [-(-min((j + 1) * _GVOC, _VOCAB) // _VB) for j in range(4)]
_NBJ = [_BEND[j] - _BSTART[j] for j in range(4)]
_NE_J = _BATCH * 8
_PW_E = _NE_J // _NW
_CH_E = 4096


def _fold_body(in_ref, out_ref, scr_ref):
    scr_ref[:, 0:16] = in_ref[...].T
    for s in range(8):
        out_ref[:, s * 16:(s + 1) * 16] = scr_ref[pl.Slice(s, _VB // 8, 8), 0:16]


def _fold_slab(embT, j):
    bs = _BSTART[j]
    return pl.pallas_call(
        _fold_body,
        grid=(_NBJ[j],),
        in_specs=[pl.BlockSpec((16, _VB), lambda i: (0, bs + i))],
        out_specs=pl.BlockSpec((_VB // 8, 128), lambda i: (i, 0)),
        out_shape=jax.ShapeDtypeStruct((_NBJ[j] * _VB // 8, 128), jnp.float32),
        scratch_shapes=[pltpu.VMEM((_VB, 128), jnp.float32)],
    )(embT)


_FCB = 65536
_NBF = (_VOCAB + _FCB - 1) // _FCB


def _fc_fold_body(in_ref, out_ref):
    out_ref[...] = in_ref[...].reshape(_FCB // 128, 128)


def _fc_fold(fcT):
    return pl.pallas_call(
        _fc_fold_body,
        grid=(_NBF,),
        in_specs=[pl.BlockSpec((1, _FCB), lambda i: (0, i))],
        out_specs=pl.BlockSpec((_FCB // 128, 128), lambda i: (i, 0)),
        out_shape=jax.ShapeDtypeStruct((_NBF * _FCB // 128, 128), jnp.float32),
    )(fcT)


def _sc_gather_slab(table, idxp_all, j, with_fc, fc1=None, idxf=None):
    mesh = plsc.VectorSubcoreMesh(core_axis_name="c", subcore_axis_name="s")
    out_type = [jax.ShapeDtypeStruct((_NE_J, _EMBED_DIM), jnp.float32)]
    scratch = [
        pltpu.VMEM((_CH_E,), jnp.int32),
        pltpu.VMEM((_CH_E, _EMBED_DIM), jnp.float32),
        pltpu.SemaphoreType.DMA,
    ]
    if with_fc:
        out_type.append(jax.ShapeDtypeStruct((_N_FC,), jnp.float32))
        scratch += [
            pltpu.VMEM((_CH_F,), jnp.int32),
            pltpu.VMEM((_CH_F,), jnp.float32),
            pltpu.SemaphoreType.DMA,
        ]

    @functools.partial(
        pl.kernel,
        out_type=tuple(out_type),
        name="sc_gather_fc" if with_fc else "sc_gather",
        mesh=mesh,
        scratch_types=scratch,
        compiler_params=pltpu.CompilerParams(use_tc_tiling_on_sc=False),
    )
    def k(*refs):
        if with_fc:
            (emb_hbm, fc_hbm, idxp_hbm, idxf_hbm, ex_hbm, fcg_hbm,
             idxe_v, rows_v, s1, idxf_v, fcr_v, s2) = refs
        else:
            emb_hbm, idxp_hbm, ex_hbm, idxe_v, rows_v, s1 = refs
        wid = lax.axis_index("s") * 2 + lax.axis_index("c")
        st = j * _NE_J + wid * _PW_E
        pltpu.sync_copy(idxp_hbm.at[pl.ds(st, _CH_E)], idxe_v)
        pltpu.async_copy(emb_hbm.at[idxe_v], rows_v, s1).wait()
        pltpu.sync_copy(rows_v, ex_hbm.at[pl.ds(wid * _PW_E, _CH_E)])
        if with_fc:
            def body_f(t, carry):
                stf = wid * _PW_F + t * _CH_F
                pltpu.sync_copy(idxf_hbm.at[pl.ds(stf, _CH_F)], idxf_v)
                pltpu.async_copy(fc_hbm.at[idxf_v], fcr_v, s2).wait()
                pltpu.sync_copy(fcr_v, fcg_hbm.at[pl.ds(stf, _CH_F)])
                return carry
            lax.fori_loop(0, _PW_F // _CH_F, body_f, 0)

    if with_fc:
        return k(table, fc1, idxp_all, idxf)
    return k(table, idxp_all)[0]


def _tc_body(e0_ref, e1_ref, e2_ref, e3_ref, fcg_ref, sp_ref, w1_ref, b1_ref,
             w2_ref, b2_ref, w3_ref, cb_ref, out_ref):
    exs = [e0_ref[...], e1_ref[...], e2_ref[...], e3_ref[...]]
    rowsum = jnp.zeros((_BS, _EMBED_DIM), jnp.float32)
    y = jnp.zeros((_BS, 128), jnp.float32)
    ssq = jnp.zeros((_BS,), jnp.float32)
    for j in range(4):
        exj = exs[j]
        y = y + jnp.dot(exj, w1_ref[j * 128:(j + 1) * 128, :],
                        preferred_element_type=jnp.float32)
        rowsum = rowsum + jnp.dot(exj, sp_ref[j * 128:(j + 1) * 128, :],
                                  preferred_element_type=jnp.float32)
        if j < 3:
            ssq = ssq + jnp.sum(exj * exj, axis=1)
        else:
            ex3 = exj[:, :32]
            ssq = ssq + jnp.sum(ex3 * ex3, axis=1)
    fm = 0.5 * (jnp.sum(rowsum * rowsum, axis=1) - ssq)
    lin = jnp.sum(fcg_ref[...], axis=1)
    h1 = jnp.maximum(y + b1_ref[...], 0.0)
    h2 = jnp.maximum(
        jnp.dot(h1, w2_ref[...], preferred_element_type=jnp.float32)
        + b2_ref[...], 0.0)
    mlp = jnp.sum(h2 * w3_ref[...], axis=1)
    out_ref[...] = lin + fm + mlp + cb_ref[0, 0]


_SEL = np.zeros((512, _EMBED_DIM), np.float32)
for _f in range(_NUM_FIELDS):
    for _d in range(_EMBED_DIM):
        _SEL[(_f // 8) * 128 + (_f % 8) * 16 + _d, _d] = 1.0


def _tc_compute(exs, fcg, W1p, b1, W2, b2, w3, cb):
    eb = pl.BlockSpec((_BS * 8 // 128 * 16, 128), lambda i: (i, 0))
    return pl.pallas_call(
        _tc_body,
        grid=(_NBLK,),
        in_specs=[
            pl.BlockSpec((_BS, 128), lambda i: (i, 0)),
            pl.BlockSpec((_BS, 128), lambda i: (i, 0)),
            pl.BlockSpec((_BS, 128), lambda i: (i, 0)),
            pl.BlockSpec((_BS, 128), lambda i: (i, 0)),
            pl.BlockSpec((_BS, _NUM_FIELDS), lambda i: (i, 0)),
            pl.BlockSpec((512, _EMBED_DIM), lambda i: (0, 0)),
            pl.BlockSpec((512, 128), lambda i: (0, 0)),
            pl.BlockSpec((1, 128), lambda i: (0, 0)),
            pl.BlockSpec((128, 64), lambda i: (0, 0)),
            pl.BlockSpec((1, 64), lambda i: (0, 0)),
            pl.BlockSpec((1, 64), lambda i: (0, 0)),
            pl.BlockSpec((1, 1), lambda i: (0, 0)),
        ],
        out_specs=pl.BlockSpec((_BS,), lambda i: (i,)),
        out_shape=jax.ShapeDtypeStruct((_BATCH,), jnp.float32),
    )(*exs, fcg, jnp.asarray(_SEL), W1p, b1, W2, b2, w3, cb)


def kernel(x, emb, fc, bias, W1, b1, W2, b2, W3, b3):
    idx = x.astype(jnp.int32) + jnp.asarray(_OFFSETS, jnp.int32)[None, :]
    idxf = idx.reshape(-1)
    embT = emb.T
    fc1 = _fc_fold(fc.T).reshape(-1)
    padded = jnp.concatenate(
        [idx, idx[:, 24:26], idx[:, 24:26], idx[:, 24:26]], axis=1)
    slot_off = np.zeros((32,), np.int32)
    for c in range(32):
        slot_off[c] = _BSTART[min(c, 31) // 8 if c < 26 else 3] * _VB
    padded = padded - jnp.asarray(slot_off)[None, :]
    idxp_all = padded.reshape(_BATCH, 4, 8).transpose(1, 0, 2).reshape(-1)
    exs = []
    fcg_flat = None
    for j in range(4):
        table_j = _fold_slab(embT, j).reshape(_NBJ[j] * _VB, _EMBED_DIM)
        if j == 0:
            ex_j, fcg_flat = _sc_gather_slab(table_j, idxp_all, j, True,
                                             fc1, idxf)
        else:
            ex_j = _sc_gather_slab(table_j, idxp_all, j, False)
        exs.append(ex_j.reshape(_NE_J // 8, 128))
    fcg = fcg_flat.reshape(_BATCH, _NUM_FIELDS)
    W1p = jnp.concatenate(
        [W1.reshape(_NUM_FIELDS, _EMBED_DIM, 128),
         jnp.zeros((32 - _NUM_FIELDS, _EMBED_DIM, 128), jnp.float32)],
        axis=0).reshape(512, 128)
    cb = (bias + b3).reshape(1, 1)
    return _tc_compute(exs, fcg, W1p, b1.reshape(1, 128), W2,
                       b2.reshape(1, 64), W3.reshape(1, 64), cb)


# speedup vs baseline: 1.2191x; 1.0010x over previous
import functools

import jax
import jax.numpy as jnp
import numpy as np
from jax import lax
from jax.experimental import pallas as pl
from jax.experimental.pallas import tpu as pltpu
from jax.experimental.pallas import tpu_sc as plsc

_FIELD_DIMS = [38462] * 26
_NUM_FIELDS = 26
_VOCAB = sum(_FIELD_DIMS)
_EMBED_DIM = 16
_BATCH = 16384
_MLP_IN = _NUM_FIELDS * _EMBED_DIM
_OFFSETS = np.concatenate(([0], np.cumsum(_FIELD_DIMS)[:-1])).astype(np.int32)

_NW = 32
_N_FC = _BATCH * _NUM_FIELDS
_PW_F = _N_FC // _NW
_CH_F = 3328
_BS = 1024
_NBLK = _BATCH // _BS

_VB = 8192
_GVOC = 8 * 38462
_BSTART = [(j * _GVOC) // _VB for j in range(4)]
_BEND = [-(-min((j + 1) * _GVOC, _VOCAB) // _VB) for j in range(4)]
_NBJ = [_BEND[j] - _BSTART[j] for j in range(4)]
_NE_J = _BATCH * 8
_PW_E = _NE_J // _NW
_CH_E = 4096


def _fold_body(in_ref, out_ref, scr_ref):
    scr_ref[:, 0:16] = in_ref[...].T
    for s in range(8):
        out_ref[:, s * 16:(s + 1) * 16] = scr_ref[pl.Slice(s, _VB // 8, 8), 0:16]


def _fold_slab(embT, j):
    bs = _BSTART[j]
    return pl.pallas_call(
        _fold_body,
        grid=(_NBJ[j],),
        in_specs=[pl.BlockSpec((16, _VB), lambda i: (0, bs + i))],
        out_specs=pl.BlockSpec((_VB // 8, 128), lambda i: (i, 0)),
        out_shape=jax.ShapeDtypeStruct((_NBJ[j] * _VB // 8, 128), jnp.float32),
        scratch_shapes=[pltpu.VMEM((_VB, 128), jnp.float32)],
    )(embT)


_FCB = 65536
_NBF = (_VOCAB + _FCB - 1) // _FCB


def _fc_fold_body(in_ref, out_ref):
    out_ref[...] = in_ref[...].reshape(_FCB // 128, 128)


def _fc_fold(fcT):
    return pl.pallas_call(
        _fc_fold_body,
        grid=(_NBF,),
        in_specs=[pl.BlockSpec((1, _FCB), lambda i: (0, i))],
        out_specs=pl.BlockSpec((_FCB // 128, 128), lambda i: (i, 0)),
        out_shape=jax.ShapeDtypeStruct((_NBF * _FCB // 128, 128), jnp.float32),
    )(fcT)


def _sc_gather_slab(table, idxp_all, j, with_fc, fc1=None, idxf=None):
    mesh = plsc.VectorSubcoreMesh(core_axis_name="c", subcore_axis_name="s")
    out_type = [jax.ShapeDtypeStruct((_NE_J, _EMBED_DIM), jnp.float32)]
    scratch = [
        pltpu.VMEM((_CH_E,), jnp.int32),
        pltpu.VMEM((_CH_E, _EMBED_DIM), jnp.float32),
        pltpu.SemaphoreType.DMA,
    ]
    if with_fc:
        out_type.append(jax.ShapeDtypeStruct((_N_FC,), jnp.float32))
        scratch += [
            pltpu.VMEM((_CH_F,), jnp.int32),
            pltpu.VMEM((_CH_F,), jnp.float32),
            pltpu.SemaphoreType.DMA,
        ]

    @functools.partial(
        pl.kernel,
        out_type=tuple(out_type),
        name="sc_gather_fc" if with_fc else "sc_gather",
        mesh=mesh,
        scratch_types=scratch,
        compiler_params=pltpu.CompilerParams(use_tc_tiling_on_sc=False),
    )
    def k(*refs):
        if with_fc:
            (emb_hbm, fc_hbm, idxp_hbm, idxf_hbm, ex_hbm, fcg_hbm,
             idxe_v, rows_v, s1, idxf_v, fcr_v, s2) = refs
        else:
            emb_hbm, idxp_hbm, ex_hbm, idxe_v, rows_v, s1 = refs
        wid = lax.axis_index("s") * 2 + lax.axis_index("c")
        st = j * _NE_J + wid * _PW_E
        pltpu.sync_copy(idxp_hbm.at[pl.ds(st, _CH_E)], idxe_v)
        pltpu.async_copy(emb_hbm.at[idxe_v], rows_v, s1).wait()
        pltpu.sync_copy(rows_v, ex_hbm.at[pl.ds(wid * _PW_E, _CH_E)])
        if with_fc:
            def body_f(t, carry):
                stf = wid * _PW_F + t * _CH_F
                pltpu.sync_copy(idxf_hbm.at[pl.ds(stf, _CH_F)], idxf_v)
                pltpu.async_copy(fc_hbm.at[idxf_v], fcr_v, s2).wait()
                pltpu.sync_copy(fcr_v, fcg_hbm.at[pl.ds(stf, _CH_F)])
                return carry
            lax.fori_loop(0, _PW_F // _CH_F, body_f, 0)

    if with_fc:
        return k(table, fc1, idxp_all, idxf)
    return k(table, idxp_all)[0]


def _tc_body(e0_ref, e1_ref, e2_ref, e3_ref, fcg_ref, sp_ref, w1_ref, b1_ref,
             w2_ref, b2_ref, w3_ref, cb_ref, out_ref):
    exs = [e0_ref[...], e1_ref[...], e2_ref[...], e3_ref[...]]
    rowsum = jnp.zeros((_BS, _EMBED_DIM), jnp.float32)
    y = jnp.zeros((_BS, 128), jnp.float32)
    ssq = jnp.zeros((_BS,), jnp.float32)
    for j in range(4):
        exj = exs[j]
        y = y + jnp.dot(exj, w1_ref[j * 128:(j + 1) * 128, :],
                        preferred_element_type=jnp.float32)
        rowsum = rowsum + jnp.dot(exj, sp_ref[j * 128:(j + 1) * 128, :],
                                  preferred_element_type=jnp.float32)
        if j < 3:
            ssq = ssq + jnp.sum(exj * exj, axis=1)
        else:
            ex3 = exj[:, :32]
            ssq = ssq + jnp.sum(ex3 * ex3, axis=1)
    fm = 0.5 * (jnp.sum(rowsum * rowsum, axis=1) - ssq)
    lin = jnp.sum(fcg_ref[...], axis=1)
    h1 = jnp.maximum(y + b1_ref[...], 0.0)
    h2 = jnp.maximum(
        jnp.dot(h1, w2_ref[...], preferred_element_type=jnp.float32)
        + b2_ref[...], 0.0)
    mlp = jnp.sum(h2 * w3_ref[...], axis=1)
    out_ref[...] = lin + fm + mlp + cb_ref[0, 0]


_SEL = np.zeros((512, _EMBED_DIM), np.float32)
for _f in range(_NUM_FIELDS):
    for _d in range(_EMBED_DIM):
        _SEL[(_f // 8) * 128 + (_f % 8) * 16 + _d, _d] = 1.0


def _tc_compute(exs, fcg, W1p, b1, W2, b2, w3, cb):
    return pl.pallas_call(
        _tc_body,
        grid=(_NBLK,),
        in_specs=[
            pl.BlockSpec((_BS, 128), lambda i: (i, 0)),
            pl.BlockSpec((_BS, 128), lambda i: (i, 0)),
            pl.BlockSpec((_BS, 128), lambda i: (i, 0)),
            pl.BlockSpec((_BS, 128), lambda i: (i, 0)),
            pl.BlockSpec((_BS, _NUM_FIELDS), lambda i: (i, 0)),
            pl.BlockSpec((512, _EMBED_DIM), lambda i: (0, 0)),
            pl.BlockSpec((512, 128), lambda i: (0, 0)),
            pl.BlockSpec((1, 128), lambda i: (0, 0)),
            pl.BlockSpec((128, 64), lambda i: (0, 0)),
            pl.BlockSpec((1, 64), lambda i: (0, 0)),
            pl.BlockSpec((1, 64), lambda i: (0, 0)),
            pl.BlockSpec((1, 1), lambda i: (0, 0)),
        ],
        out_specs=pl.BlockSpec((_BS,), lambda i: (i,)),
        out_shape=jax.ShapeDtypeStruct((_BATCH,), jnp.float32),
    )(*exs, fcg, jnp.asarray(_SEL), W1p, b1, W2, b2, w3, cb)


def kernel(x, emb, fc, bias, W1, b1, W2, b2, W3, b3):
    idx = x.astype(jnp.int32) + jnp.asarray(_OFFSETS, jnp.int32)[None, :]
    idxf = idx.reshape(-1)
    embT = emb.T
    fc1 = _fc_fold(fc.T).reshape(-1)
    padded = jnp.concatenate(
        [idx, idx[:, 24:26], idx[:, 24:26], idx[:, 24:26]], axis=1)
    slot_off = np.zeros((32,), np.int32)
    for c in range(32):
        slot_off[c] = _BSTART[min(c, 31) // 8 if c < 26 else 3] * _VB
    padded = padded - jnp.asarray(slot_off)[None, :]
    idxp_all = padded.reshape(_BATCH, 4, 8).transpose(1, 0, 2).reshape(-1)
    exs = []
    fcg_flat = None
    for j in range(4):
        table_j = _fold_slab(embT, j).reshape(_NBJ[j] * _VB, _EMBED_DIM)
        if j == 0:
            ex_j, fcg_flat = _sc_gather_slab(table_j, idxp_all, j, True,
                                             fc1, idxf)
        else:
            ex_j = _sc_gather_slab(table_j, idxp_all, j, False)
        exs.append(ex_j.reshape(_NE_J // 8, 128))
    fcg = fcg_flat.reshape(_BATCH, _NUM_FIELDS)
    W1p = jnp.concatenate(
        [W1.reshape(_NUM_FIELDS, _EMBED_DIM, 128),
         jnp.zeros((32 - _NUM_FIELDS, _EMBED_DIM, 128), jnp.float32)],
        axis=0).reshape(512, 128)
    cb = (bias + b3).reshape(1, 1)
    return _tc_compute(exs, fcg, W1p, b1.reshape(1, 128), W2,
                       b2.reshape(1, 64), W3.reshape(1, 64), cb)
